# baseline (speedup 1.0000x reference)
import jax
import jax.numpy as jnp
from jax import lax
from jax.experimental import pallas as pl
from jax.experimental.pallas import tpu as pltpu
from jax.experimental.pallas import tpu_sc as plsc

N_SEG = 10000
E = 320000
D = 128
NV = D // 16

SC_SEG = 6288
TC_SEG = N_SEG - SC_SEG
TC_TILE = 64
TC_GRID = TC_SEG // TC_TILE
TC_CHUNK = 256

NW = 32
NW_BIG = 18
SEG_A = 200
SEG_B = 192
CHUNK = 320
ILEN = 216


def _sc_body(x_hbm, iptr_hbm, out_hbm, iptr_v, rowbuf, outbuf, sems):
    cid = lax.axis_index("c")
    sid = lax.axis_index("s")
    w = sid * 2 + cid
    seg_lo = w * SEG_B + 8 * jnp.minimum(w, NW_BIG)
    nseg = jnp.where(w < NW_BIG, SEG_A, SEG_B)
    pltpu.sync_copy(iptr_hbm.at[pl.ds(seg_lo, ILEN)], iptr_v)

    row_lo = iptr_v[pl.ds(0, 16)][0]
    row_hi = iptr_v[pl.ds(nseg, 16)][0]
    start0 = jnp.minimum((row_lo // 8) * 8, E - CHUNK)
    nchunks = jnp.maximum((row_hi - start0 + CHUNK - 1) // CHUNK, 1)

    def chunk_start(c):
        return jnp.minimum(start0 + c * CHUNK, E - CHUNK)

    def dma(c):
        b = lax.rem(c, 2)
        return pltpu.make_async_copy(
            x_hbm.at[pl.ds(chunk_start(c), CHUNK)], rowbuf.at[b], sems.at[b])

    dma(jnp.int32(0)).start()

    @pl.when(nchunks > 1)
    def _():
        dma(jnp.int32(1)).start()

    dma(jnp.int32(0)).wait()

    zeros = tuple(jnp.zeros((16,), jnp.float32) for _ in range(NV))

    nsteps = nseg + nchunks - 1

    def step(_, st):
        r, s_cur, c = st[0], st[1], st[2]
        accs = st[3:]
        bounds = iptr_v[pl.ds(s_cur, 16)]
        lo = bounds[0]
        hi = bounds[1]
        r_end = jnp.minimum(row_hi, start0 + (c + 1) * CHUNK)
        run_end = jnp.minimum(hi, r_end)
        b = lax.rem(c, 2)
        sc = chunk_start(c)

        def row_body(rr, a):
            o = rr - sc
            return tuple(a[j] + rowbuf[b, o, pl.ds(j * 16, 16)]
                         for j in range(NV))

        accs = plsc.parallel_loop(r, run_end, unroll=4,
                                  carry=accs)(row_body)

        fin = hi <= r_end

        @pl.when(fin)
        def _():
            nvec = jnp.full((16,), 1.0, jnp.float32) * (
                jnp.maximum(hi - lo, 1).astype(jnp.float32))
            scale = jnp.full((16,), 1.0, jnp.float32) / nvec
            for j in range(NV):
                outbuf[s_cur, pl.ds(j * 16, 16)] = accs[j] * scale

        @pl.when(jnp.logical_not(fin))
        def _():
            dma(c + 1).wait()

            @pl.when(c + 2 < nchunks)
            def _():
                dma(c + 2).start()

        s_next = jnp.where(fin, s_cur + 1, s_cur)
        c_next = jnp.where(fin, c, c + 1)
        accs = tuple(
            jnp.where(fin, jnp.zeros((16,), jnp.float32), a)
            for a in accs)
        return (run_end, s_next, c_next) + accs

    lax.fori_loop(0, nsteps, step,
                  (row_lo, jnp.int32(0), jnp.int32(0)) + zeros)

    @pl.when(w < NW_BIG)
    def _():
        pltpu.sync_copy(outbuf.at[pl.ds(0, SEG_A)],
                        out_hbm.at[pl.ds(seg_lo, SEG_A)])

    @pl.when(w >= NW_BIG)
    def _():
        pltpu.sync_copy(outbuf.at[pl.ds(0, SEG_B)],
                        out_hbm.at[pl.ds(seg_lo, SEG_B)])


def _sc_means(x, iptr):
    mesh = plsc.VectorSubcoreMesh(core_axis_name="c", subcore_axis_name="s")
    f = pl.kernel(
        _sc_body,
        mesh=mesh,
        out_type=jax.ShapeDtypeStruct((SC_SEG, D), jnp.float32),
        scratch_types=[
            pltpu.VMEM((ILEN,), jnp.int32),
            pltpu.VMEM((2, CHUNK, D), jnp.float32),
            pltpu.VMEM((SEG_A, D), jnp.float32),
            pltpu.SemaphoreType.DMA((2,)),
        ],
    )
    return f(x, iptr)


def _tc_body(x_hbm, lo_ref, hi_ref, o_ref, rowbuf, sems):
    lo_vec = lo_ref[0, 0]
    hi_vec = hi_ref[0, 0]
    lo_t = lo_vec[0]
    hi_t = hi_vec[TC_TILE - 1]
    base = jnp.minimum((lo_t // 8) * 8, E - TC_CHUNK)
    nch = jnp.maximum((hi_t - base + TC_CHUNK - 1) // TC_CHUNK, 0)

    def dma(c):
        b = lax.rem(c, 2)
        st = jnp.minimum(base + c * TC_CHUNK, E - TC_CHUNK)
        return pltpu.make_async_copy(
            x_hbm.at[pl.ds(st, TC_CHUNK)], rowbuf.at[b], sems.at[b])

    @pl.when(nch > 0)
    def _():
        dma(jnp.int32(0)).start()

    @pl.when(nch > 1)
    def _():
        dma(jnp.int32(1)).start()

    def chunk(c, acc):
        b = lax.rem(c, 2)
        canon_lo = base + c * TC_CHUNK
        st = jnp.minimum(canon_lo, E - TC_CHUNK)
        dma(c).wait()

        @pl.when(c + 2 < nch)
        def _():
            dma(c + 2).start()

        rows = rowbuf[b]
        riota = st + lax.broadcasted_iota(jnp.int32, (TC_CHUNK, 1), 0)
        lo_eff = jnp.maximum(lo_vec, canon_lo)
        member = (riota >= lo_eff[None, :]) & (riota < hi_vec[None, :])
        pmat = member.astype(jnp.bfloat16)
        return acc + lax.dot_general(
            pmat, rows.astype(jnp.bfloat16), (((0,), (0,)), ((), ())),
            preferred_element_type=jnp.float32)

    acc = lax.fori_loop(0, nch, chunk, jnp.zeros((TC_TILE, D), jnp.float32))
    denom = jnp.maximum((hi_vec - lo_vec).astype(jnp.float32), 1.0)
    o_ref[...] = acc / denom[:, None]


def _tc_means(x, iptr):
    lo3 = lax.slice(iptr, (SC_SEG,), (N_SEG,)).reshape(TC_GRID, 1, TC_TILE)
    hi3 = lax.slice(iptr, (SC_SEG + 1,), (N_SEG + 1,)).reshape(
        TC_GRID, 1, TC_TILE)
    return pl.pallas_call(
        _tc_body,
        grid=(TC_GRID,),
        in_specs=[
            pl.BlockSpec(memory_space=pl.ANY),
            pl.BlockSpec((1, 1, TC_TILE), lambda t: (t, 0, 0)),
            pl.BlockSpec((1, 1, TC_TILE), lambda t: (t, 0, 0)),
        ],
        out_specs=pl.BlockSpec((TC_TILE, D), lambda t: (t, 0)),
        out_shape=jax.ShapeDtypeStruct((TC_SEG, D), jnp.float32),
        scratch_shapes=[
            pltpu.VMEM((2, TC_CHUNK, D), jnp.float32),
            pltpu.SemaphoreType.DMA((2,)),
        ],
    )(x, lo3, hi3)


def kernel(x, indptr):
    iptr = indptr.astype(jnp.int32)
    sc_out = _sc_means(x, iptr)
    tc_out = _tc_means(x, iptr)
    return jnp.concatenate([sc_out, tc_out], axis=0)


# speedup vs baseline: 2.7201x; 2.7201x over previous
import jax
import jax.numpy as jnp
from jax import lax
from jax.experimental import pallas as pl
from jax.experimental.pallas import tpu as pltpu
from jax.experimental.pallas import tpu_sc as plsc

N_SEG = 10000
E = 320000
D = 128
NV = D // 16

SC_SEG = 7184
TC_SEG = N_SEG - SC_SEG
TC_TILE = 128
TC_GRID = TC_SEG // TC_TILE
TC_CHUNK = 512
TC_NBUF = 6

NW = 32
NW_BIG = 2
SEG_A = 232
SEG_B = 224
CHUNK = 320
ILEN = 248


def _sc_body(x_hbm, iptr_hbm, out_hbm, iptr_v, rowbuf, outbuf, sems):
    cid = lax.axis_index("c")
    sid = lax.axis_index("s")
    w = sid * 2 + cid
    seg_lo = w * SEG_B + 8 * jnp.minimum(w, NW_BIG)
    nseg = jnp.where(w < NW_BIG, SEG_A, SEG_B)
    pltpu.sync_copy(iptr_hbm.at[pl.ds(seg_lo, ILEN)], iptr_v)

    row_lo = iptr_v[pl.ds(0, 16)][0]
    row_hi = iptr_v[pl.ds(nseg, 16)][0]
    start0 = jnp.minimum((row_lo // 8) * 8, E - CHUNK)
    nchunks = jnp.maximum((row_hi - start0 + CHUNK - 1) // CHUNK, 1)

    def chunk_start(c):
        return jnp.minimum(start0 + c * CHUNK, E - CHUNK)

    def dma(c):
        b = lax.rem(c, 2)
        return pltpu.make_async_copy(
            x_hbm.at[pl.ds(chunk_start(c), CHUNK)], rowbuf.at[b], sems.at[b])

    dma(jnp.int32(0)).start()

    @pl.when(nchunks > 1)
    def _():
        dma(jnp.int32(1)).start()

    dma(jnp.int32(0)).wait()

    zeros = tuple(jnp.zeros((16,), jnp.float32) for _ in range(NV))

    nsteps = nseg + nchunks - 1

    def step(_, st):
        r, s_cur, c = st[0], st[1], st[2]
        accs = st[3:]
        bounds = iptr_v[pl.ds(s_cur, 16)]
        lo = bounds[0]
        hi = bounds[1]
        r_end = jnp.minimum(row_hi, start0 + (c + 1) * CHUNK)
        run_end = jnp.minimum(hi, r_end)
        b = lax.rem(c, 2)
        sc = chunk_start(c)

        def row_body(rr, a):
            o = rr - sc
            return tuple(a[j] + rowbuf[b, o, pl.ds(j * 16, 16)]
                         for j in range(NV))

        accs = plsc.parallel_loop(r, run_end, unroll=4,
                                  carry=accs)(row_body)

        fin = hi <= r_end

        @pl.when(fin)
        def _():
            nvec = jnp.full((16,), 1.0, jnp.float32) * (
                jnp.maximum(hi - lo, 1).astype(jnp.float32))
            scale = jnp.full((16,), 1.0, jnp.float32) / nvec
            for j in range(NV):
                outbuf[s_cur, pl.ds(j * 16, 16)] = accs[j] * scale

        @pl.when(jnp.logical_not(fin))
        def _():
            dma(c + 1).wait()

            @pl.when(c + 2 < nchunks)
            def _():
                dma(c + 2).start()

        s_next = jnp.where(fin, s_cur + 1, s_cur)
        c_next = jnp.where(fin, c, c + 1)
        accs = tuple(
            jnp.where(fin, jnp.zeros((16,), jnp.float32), a)
            for a in accs)
        return (run_end, s_next, c_next) + accs

    lax.fori_loop(0, nsteps, step,
                  (row_lo, jnp.int32(0), jnp.int32(0)) + zeros)

    @pl.when(w < NW_BIG)
    def _():
        pltpu.sync_copy(outbuf.at[pl.ds(0, SEG_A)],
                        out_hbm.at[pl.ds(seg_lo, SEG_A)])

    @pl.when(w >= NW_BIG)
    def _():
        pltpu.sync_copy(outbuf.at[pl.ds(0, SEG_B)],
                        out_hbm.at[pl.ds(seg_lo, SEG_B)])


def _sc_means(x, iptr):
    mesh = plsc.VectorSubcoreMesh(core_axis_name="c", subcore_axis_name="s")
    f = pl.kernel(
        _sc_body,
        mesh=mesh,
        out_type=jax.ShapeDtypeStruct((SC_SEG, D), jnp.float32),
        scratch_types=[
            pltpu.VMEM((ILEN,), jnp.int32),
            pltpu.VMEM((2, CHUNK, D), jnp.float32),
            pltpu.VMEM((SEG_A, D), jnp.float32),
            pltpu.SemaphoreType.DMA((2,)),
        ],
    )
    return f(x, iptr)


def _tc_body(x_hbm, lo_ref, hi_ref, o_ref, rowbuf, sems):
    lo_vec = lo_ref[0, 0]
    hi_vec = hi_ref[0, 0]
    lo_t = lo_vec[0]
    hi_t = hi_vec[TC_TILE - 1]
    base = jnp.minimum((lo_t // 8) * 8, E - TC_CHUNK)
    nch = jnp.maximum((hi_t - base + TC_CHUNK - 1) // TC_CHUNK, 0)
    npair = (nch + 1) // 2
    nissue = 2 * npair

    def dma(c):
        b = lax.rem(c, TC_NBUF)
        st = jnp.minimum(base + c * TC_CHUNK, E - TC_CHUNK)
        return pltpu.make_async_copy(
            x_hbm.at[pl.ds(st, TC_CHUNK)], rowbuf.at[b], sems.at[b])

    for k in range(4):
        @pl.when(k < nissue)
        def _(k=k):
            dma(jnp.int32(k)).start()

    def chunk(c, acc):
        b = lax.rem(c, TC_NBUF)
        canon_lo = base + c * TC_CHUNK
        st = jnp.minimum(canon_lo, E - TC_CHUNK)
        dma(c).wait()

        @pl.when(c + 4 < nissue)
        def _():
            dma(c + 4).start()

        rows = rowbuf[b]
        riota = st + lax.broadcasted_iota(jnp.int32, (TC_CHUNK, 1), 0)
        lo_eff = jnp.maximum(lo_vec, canon_lo)
        member = (riota >= lo_eff[None, :]) & (riota < hi_vec[None, :])
        pmat = member.astype(jnp.bfloat16)
        return acc + lax.dot_general(
            pmat, rows.astype(jnp.bfloat16), (((0,), (0,)), ((), ())),
            preferred_element_type=jnp.float32)

    def pair(i, accs):
        a1 = chunk(2 * i, accs[0])
        a2 = chunk(2 * i + 1, accs[1])
        return (a1, a2)

    zacc = jnp.zeros((TC_TILE, D), jnp.float32)
    a1, a2 = lax.fori_loop(0, npair, pair, (zacc, zacc))
    acc = a1 + a2
    denom = jnp.maximum((hi_vec - lo_vec).astype(jnp.float32), 1.0)
    o_ref[...] = acc / denom[:, None]


def _tc_means(x, iptr):
    lo3 = lax.slice(iptr, (SC_SEG,), (N_SEG,)).reshape(TC_GRID, 1, TC_TILE)
    hi3 = lax.slice(iptr, (SC_SEG + 1,), (N_SEG + 1,)).reshape(
        TC_GRID, 1, TC_TILE)
    return pl.pallas_call(
        _tc_body,
        grid=(TC_GRID,),
        in_specs=[
            pl.BlockSpec(memory_space=pl.ANY),
            pl.BlockSpec((1, 1, TC_TILE), lambda t: (t, 0, 0)),
            pl.BlockSpec((1, 1, TC_TILE), lambda t: (t, 0, 0)),
        ],
        out_specs=pl.BlockSpec((TC_TILE, D), lambda t: (t, 0)),
        out_shape=jax.ShapeDtypeStruct((TC_SEG, D), jnp.float32),
        scratch_shapes=[
            pltpu.VMEM((TC_NBUF, TC_CHUNK, D), jnp.float32),
            pltpu.SemaphoreType.DMA((TC_NBUF,)),
        ],
    )(x, lo3, hi3)


def kernel(x, indptr):
    iptr = indptr.astype(jnp.int32)
    sc_out = _sc_means(x, iptr)
    tc_out = _tc_means(x, iptr)
    return jnp.concatenate([sc_out, tc_out], axis=0)


# speedup vs baseline: 2.9230x; 1.0746x over previous
import jax
import jax.numpy as jnp
from jax import lax
from jax.experimental import pallas as pl
from jax.experimental.pallas import tpu as pltpu
from jax.experimental.pallas import tpu_sc as plsc

N_SEG = 10000
E = 320000
D = 128
NV = D // 16

SC_SEG = 7440
TC_SEG = N_SEG - SC_SEG
TC_TILE = 128
TC_GRID = TC_SEG // TC_TILE
TC_CHUNK = 512
TC_NBUF = 8

NW = 32
NW_BIG = 2
SEG_A = 240
SEG_B = 232
CHUNK = 320
ILEN = 256


def _sc_body(x_hbm, iptr_hbm, out_hbm, iptr_v, rowbuf, outbuf, sems):
    cid = lax.axis_index("c")
    sid = lax.axis_index("s")
    w = sid * 2 + cid
    seg_lo = w * SEG_B + 8 * jnp.minimum(w, NW_BIG)
    nseg = jnp.where(w < NW_BIG, SEG_A, SEG_B)
    pltpu.sync_copy(iptr_hbm.at[pl.ds(seg_lo, ILEN)], iptr_v)

    row_lo = iptr_v[pl.ds(0, 16)][0]
    row_hi = iptr_v[pl.ds(nseg, 16)][0]
    start0 = jnp.minimum((row_lo // 8) * 8, E - CHUNK)
    nchunks = jnp.maximum((row_hi - start0 + CHUNK - 1) // CHUNK, 1)

    def chunk_start(c):
        return jnp.minimum(start0 + c * CHUNK, E - CHUNK)

    def dma(c):
        b = lax.rem(c, 2)
        return pltpu.make_async_copy(
            x_hbm.at[pl.ds(chunk_start(c), CHUNK)], rowbuf.at[b], sems.at[b])

    dma(jnp.int32(0)).start()

    @pl.when(nchunks > 1)
    def _():
        dma(jnp.int32(1)).start()

    dma(jnp.int32(0)).wait()

    zeros = tuple(jnp.zeros((16,), jnp.float32) for _ in range(NV))

    nsteps = nseg + nchunks - 1

    def step(_, st):
        r, s_cur, c = st[0], st[1], st[2]
        accs = st[3:]
        bounds = iptr_v[pl.ds(s_cur, 16)]
        lo = bounds[0]
        hi = bounds[1]
        r_end = jnp.minimum(row_hi, start0 + (c + 1) * CHUNK)
        run_end = jnp.minimum(hi, r_end)
        b = lax.rem(c, 2)
        sc = chunk_start(c)

        def row_body(rr, a):
            o = rr - sc
            return tuple(a[j] + rowbuf[b, o, pl.ds(j * 16, 16)]
                         for j in range(NV))

        accs = plsc.parallel_loop(r, run_end, unroll=4,
                                  carry=accs)(row_body)

        fin = hi <= r_end

        @pl.when(fin)
        def _():
            nvec = jnp.full((16,), 1.0, jnp.float32) * (
                jnp.maximum(hi - lo, 1).astype(jnp.float32))
            scale = jnp.full((16,), 1.0, jnp.float32) / nvec
            for j in range(NV):
                outbuf[s_cur, pl.ds(j * 16, 16)] = accs[j] * scale

        @pl.when(jnp.logical_not(fin))
        def _():
            dma(c + 1).wait()

            @pl.when(c + 2 < nchunks)
            def _():
                dma(c + 2).start()

        s_next = jnp.where(fin, s_cur + 1, s_cur)
        c_next = jnp.where(fin, c, c + 1)
        accs = tuple(
            jnp.where(fin, jnp.zeros((16,), jnp.float32), a)
            for a in accs)
        return (run_end, s_next, c_next) + accs

    lax.fori_loop(0, nsteps, step,
                  (row_lo, jnp.int32(0), jnp.int32(0)) + zeros)

    @pl.when(w < NW_BIG)
    def _():
        pltpu.sync_copy(outbuf.at[pl.ds(0, SEG_A)],
                        out_hbm.at[pl.ds(seg_lo, SEG_A)])

    @pl.when(w >= NW_BIG)
    def _():
        pltpu.sync_copy(outbuf.at[pl.ds(0, SEG_B)],
                        out_hbm.at[pl.ds(seg_lo, SEG_B)])


def _sc_means(x, iptr):
    mesh = plsc.VectorSubcoreMesh(core_axis_name="c", subcore_axis_name="s")
    f = pl.kernel(
        _sc_body,
        mesh=mesh,
        out_type=jax.ShapeDtypeStruct((SC_SEG, D), jnp.float32),
        scratch_types=[
            pltpu.VMEM((ILEN,), jnp.int32),
            pltpu.VMEM((2, CHUNK, D), jnp.float32),
            pltpu.VMEM((SEG_A, D), jnp.float32),
            pltpu.SemaphoreType.DMA((2,)),
        ],
    )
    return f(x, iptr)


def _tc_body(x_hbm, lo_ref, hi_ref, o_ref, rowbuf, sems):
    lo_vec = lo_ref[0, 0]
    hi_vec = hi_ref[0, 0]
    lo_t = lo_vec[0]
    hi_t = hi_vec[TC_TILE - 1]
    base = jnp.minimum((lo_t // 8) * 8, E - TC_CHUNK)
    nch = jnp.maximum((hi_t - base + TC_CHUNK - 1) // TC_CHUNK, 0)
    nquad = (nch + 3) // 4
    nissue = 4 * nquad

    def dma(c):
        b = lax.rem(c, TC_NBUF)
        st = jnp.minimum(base + c * TC_CHUNK, E - TC_CHUNK)
        return pltpu.make_async_copy(
            x_hbm.at[pl.ds(st, TC_CHUNK)], rowbuf.at[b], sems.at[b])

    for k in range(6):
        @pl.when(k < nissue)
        def _(k=k):
            dma(jnp.int32(k)).start()

    def chunk(c, acc):
        b = lax.rem(c, TC_NBUF)
        canon_lo = base + c * TC_CHUNK
        st = jnp.minimum(canon_lo, E - TC_CHUNK)
        dma(c).wait()

        @pl.when(c + 6 < nissue)
        def _():
            dma(c + 6).start()

        rows = rowbuf[b]
        riota = st + lax.broadcasted_iota(jnp.int32, (TC_CHUNK, 1), 0)
        lo_eff = jnp.maximum(lo_vec, canon_lo)
        member = (riota >= lo_eff[None, :]) & (riota < hi_vec[None, :])
        pmat = member.astype(jnp.bfloat16)
        return acc + lax.dot_general(
            pmat, rows.astype(jnp.bfloat16), (((0,), (0,)), ((), ())),
            preferred_element_type=jnp.float32)

    def quad(i, accs):
        return tuple(chunk(4 * i + k, accs[k]) for k in range(4))

    zacc = jnp.zeros((TC_TILE, D), jnp.float32)
    a1, a2, a3, a4 = lax.fori_loop(0, nquad, quad, (zacc,) * 4)
    acc = (a1 + a2) + (a3 + a4)
    denom = jnp.maximum((hi_vec - lo_vec).astype(jnp.float32), 1.0)
    o_ref[...] = acc / denom[:, None]


def _tc_means(x, iptr):
    lo3 = lax.slice(iptr, (SC_SEG,), (N_SEG,)).reshape(TC_GRID, 1, TC_TILE)
    hi3 = lax.slice(iptr, (SC_SEG + 1,), (N_SEG + 1,)).reshape(
        TC_GRID, 1, TC_TILE)
    return pl.pallas_call(
        _tc_body,
        grid=(TC_GRID,),
        in_specs=[
            pl.BlockSpec(memory_space=pl.ANY),
            pl.BlockSpec((1, 1, TC_TILE), lambda t: (t, 0, 0)),
            pl.BlockSpec((1, 1, TC_TILE), lambda t: (t, 0, 0)),
        ],
        out_specs=pl.BlockSpec((TC_TILE, D), lambda t: (t, 0)),
        out_shape=jax.ShapeDtypeStruct((TC_SEG, D), jnp.float32),
        scratch_shapes=[
            pltpu.VMEM((TC_NBUF, TC_CHUNK, D), jnp.float32),
            pltpu.SemaphoreType.DMA((TC_NBUF,)),
        ],
    )(x, lo3, hi3)


def kernel(x, indptr):
    iptr = indptr.astype(jnp.int32)
    sc_out = _sc_means(x, iptr)
    tc_out = _tc_means(x, iptr)
    return jnp.concatenate([sc_out, tc_out], axis=0)
